# baseline (speedup 1.0000x reference)
import functools

import jax
import jax.numpy as jnp
from jax import lax
from jax.experimental import pallas as pl
from jax.experimental.pallas import tpu as pltpu
from jax.experimental.pallas import tpu_sc as plsc

D1, D2, D3 = 300, 64, 32
DOUT = D1 + D2 + D3
DCAT = 384
NPOS = 101
B = 16384
NC, NS = 2, 16
NW = NC * NS
BPW = B // NW
C = 128
NCHUNK = BPW // C


def _body(words_hbm, cat_hbm, pos_hbm, out_hbm,
          idx_v, pos_v, comb_v, sem1):
    cid = lax.axis_index("c")
    sid = lax.axis_index("s")
    wid = sid * NC + cid
    base = wid * BPW

    pltpu.sync_copy(words_hbm.at[pl.ds(base, BPW)], idx_v)
    pltpu.sync_copy(pos_hbm, pos_v)

    lanes = lax.iota(jnp.int32, 16)

    @pl.loop(0, NCHUNK)
    def _chunk(c):
        row0 = base + c * C
        idx_c = idx_v.at[pl.ds(c * C, C)]

        cp = pltpu.async_copy(
            cat_hbm.at[idx_c], comb_v.at[:, pl.ds(0, DCAT)], sem1)
        cp.wait()

        @pl.loop(0, C)
        def _pos(i):
            irow = jnp.full((16,), i, jnp.int32)
            pidx = jnp.minimum(row0 + i, NPOS - 1)
            prow = jnp.full((16,), pidx, jnp.int32)
            for off in range(0, D3, 16):
                t = plsc.load_gather(pos_v, [prow, off + lanes])
                plsc.store_scatter(
                    comb_v, [irow, (D1 + D2 + off) + lanes], t)

        pltpu.sync_copy(
            comb_v, out_hbm.at[pl.ds(pl.multiple_of(row0, 8), C)])


@jax.jit
def _embed(words, pretrained, learned, pos_table):
    cat = jnp.concatenate(
        [pretrained, learned,
         jnp.zeros((pretrained.shape[0], DCAT - D1 - D2), jnp.float32)],
        axis=1)
    mesh = plsc.VectorSubcoreMesh(core_axis_name="c", subcore_axis_name="s")
    f = functools.partial(
        pl.kernel,
        mesh=mesh,
        compiler_params=pltpu.CompilerParams(needs_layout_passes=False),
        out_type=jax.ShapeDtypeStruct((B, DOUT), jnp.float32),
        scratch_types=[
            pltpu.VMEM((BPW,), jnp.int32),
            pltpu.VMEM((NPOS, D3), jnp.float32),
            pltpu.VMEM((C, DOUT), jnp.float32),
            pltpu.SemaphoreType.DMA,
        ],
    )(_body)
    return f(words, cat, pos_table)


def kernel(words, pretrained, learned, pos_table):
    return _embed(words.astype(jnp.int32), pretrained, learned, pos_table)


# speedup vs baseline: 2.0236x; 2.0236x over previous
import functools

import jax
import jax.numpy as jnp
from jax import lax
from jax.experimental import pallas as pl
from jax.experimental.pallas import tpu as pltpu
from jax.experimental.pallas import tpu_sc as plsc

D1, D2, D3 = 300, 64, 32
DOUT = D1 + D2 + D3
NPOS = 101
B = 16384
NC, NS = 2, 16
NW = NC * NS
BPW = B // NW
C = 64
NCHUNK = BPW // C
G = 8
NG = C // G


def _body(words_hbm, pre_hbm, lrn_hbm, pos_hbm, out_hbm,
          idx_v, pos_v, r1a, r1b, r2a, r2b, comb_v, sub_s,
          sem1a, sem1b, sem2a, sem2b):
    cid = lax.axis_index("c")
    sid = lax.axis_index("s")
    wid = sid * NC + cid
    base = wid * BPW

    pltpu.sync_copy(words_hbm.at[pl.ds(base, BPW)], idx_v.at[pl.ds(0, BPW)])
    pltpu.sync_copy(pos_hbm, pos_v)

    lanes = lax.iota(jnp.int32, 16)
    tail_mask = lanes >= 4

    def fetch(g, r1, r2, s1, s2, buf):
        vec = idx_v[pl.ds(g * G, 16)]
        for k in range(G):
            w = vec[k]
            sub_s[buf, k] = w % 8
            blk = pl.multiple_of((w // 8) * 8, 8)
            pltpu.async_copy(
                pre_hbm.at[pl.ds(blk, 8)], r1.at[pl.ds(k * 8, 8)], s1)
            pltpu.async_copy(
                lrn_hbm.at[pl.ds(blk, 8)], r2.at[pl.ds(k * 8, 8)], s2)

    def wait(r1, r2, s1, s2):
        pltpu.make_async_copy(pre_hbm.at[pl.ds(0, G * 8)], r1, s1).wait()
        pltpu.make_async_copy(lrn_hbm.at[pl.ds(0, G * 8)], r2, s2).wait()

    def extract(g, row0, r1, r2, buf):
        @pl.loop(0, G)
        def _asm(k):
            s = k * 8 + sub_s[buf, k]
            i = (g % NG) * G + k
            for off in range(0, 288, 16):
                comb_v[i, pl.ds(off, 16)] = r1[s, pl.ds(off, 16)]
            irow = jnp.full((16,), i, jnp.int32)
            srow = jnp.full((16,), s, jnp.int32)
            t = plsc.load_gather(r1, [srow, (D1 - 16) + lanes])
            plsc.store_scatter(
                comb_v, [irow, (D1 - 16) + lanes], t, mask=tail_mask)
            for off in range(0, D2, 16):
                t2 = r2[s, pl.ds(off, 16)]
                plsc.store_scatter(comb_v, [irow, (D1 + off) + lanes], t2)
            pidx = jnp.minimum(row0 + (g % NG) * G + k, NPOS - 1)
            prow = jnp.full((16,), pidx, jnp.int32)
            for off in range(0, D3, 16):
                t3 = plsc.load_gather(pos_v, [prow, off + lanes])
                plsc.store_scatter(
                    comb_v, [irow, (D1 + D2 + off) + lanes], t3)

    fetch(0, r1a, r2a, sem1a, sem2a, 0)

    @pl.loop(0, NCHUNK)
    def _chunk(c):
        row0 = base + c * C

        @pl.loop(0, NG // 2)
        def _pair(h):
            g = c * NG + 2 * h
            wait(r1a, r2a, sem1a, sem2a)
            fetch(g + 1, r1b, r2b, sem1b, sem2b, 1)
            extract(g, row0, r1a, r2a, 0)
            wait(r1b, r2b, sem1b, sem2b)

            @pl.when(g + 2 < NCHUNK * NG)
            def _():
                fetch(g + 2, r1a, r2a, sem1a, sem2a, 0)

            extract(g + 1, row0, r1b, r2b, 1)

        pltpu.sync_copy(
            comb_v, out_hbm.at[pl.ds(pl.multiple_of(row0, 8), C)])


@jax.jit
def _embed(words, pretrained, learned, pos_table):
    mesh = plsc.VectorSubcoreMesh(core_axis_name="c", subcore_axis_name="s")
    f = functools.partial(
        pl.kernel,
        mesh=mesh,
        compiler_params=pltpu.CompilerParams(needs_layout_passes=False),
        out_type=jax.ShapeDtypeStruct((B, DOUT), jnp.float32),
        scratch_types=[
            pltpu.VMEM((BPW + 16,), jnp.int32),
            pltpu.VMEM((NPOS, D3), jnp.float32),
            pltpu.VMEM((G * 8, D1), jnp.float32),
            pltpu.VMEM((G * 8, D1), jnp.float32),
            pltpu.VMEM((G * 8, D2), jnp.float32),
            pltpu.VMEM((G * 8, D2), jnp.float32),
            pltpu.VMEM((C, DOUT), jnp.float32),
            pltpu.SMEM((2, G), jnp.int32),
            pltpu.SemaphoreType.DMA,
            pltpu.SemaphoreType.DMA,
            pltpu.SemaphoreType.DMA,
            pltpu.SemaphoreType.DMA,
        ],
    )(_body)
    return f(words, pretrained, learned, pos_table)


def kernel(words, pretrained, learned, pos_table):
    return _embed(words.astype(jnp.int32), pretrained, learned, pos_table)
